# baseline (speedup 1.0000x reference)
import functools

import jax
import jax.numpy as jnp
from jax import lax
from jax.experimental import pallas as pl
from jax.experimental.pallas import tpu as pltpu
from jax.experimental.pallas import tpu_sc as plsc

VOCAB = 1_000_000
D = 64
BATCH = 4096
SEQ = 50
SPLIT = 4
DS = D // SPLIT
B = BATCH * SEQ * SPLIT

NC = 2
NS = 16
NW = NC * NS
B_PER_W = B // NW
CHUNK = 512
NCHUNK = B_PER_W // CHUNK
NBUF = 5
LOOKAHEAD = NBUF - 1

_mesh = plsc.VectorSubcoreMesh(
    core_axis_name="c", subcore_axis_name="s", num_cores=NC, num_subcores=NS
)


@functools.partial(
    pl.kernel,
    out_type=jax.ShapeDtypeStruct((B, DS), jnp.float32),
    mesh=_mesh,
    scratch_types=[
        pltpu.VMEM((B_PER_W,), jnp.int32),
        pltpu.VMEM((NBUF, CHUNK, DS), jnp.float32),
        pltpu.SemaphoreType.DMA((NBUF,)),
        pltpu.SemaphoreType.DMA((NBUF,)),
    ],
    compiler_params=pltpu.CompilerParams(use_tc_tiling_on_sc=False),
)
def _gather_kernel(idx_hbm, table_hbm, out_hbm, idx_v, rows_v, gsem, osem):
    wid = lax.axis_index("s") * NC + lax.axis_index("c")
    base = wid * B_PER_W

    pltpu.sync_copy(idx_hbm.at[pl.ds(base, B_PER_W)], idx_v)

    def gather(c, slot):
        return pltpu.make_async_copy(
            table_hbm.at[idx_v.at[pl.ds(c * CHUNK, CHUNK)]],
            rows_v.at[slot],
            gsem.at[slot],
        )

    def write(c, slot):
        return pltpu.make_async_copy(
            rows_v.at[slot],
            out_hbm.at[pl.ds(base + c * CHUNK, CHUNK)],
            osem.at[slot],
        )

    for c in range(LOOKAHEAD):
        gather(c, c).start()

    def step(c, b, first=False, issue=True):
        if issue:
            g_slot = (b + LOOKAHEAD) % NBUF
            if not first:
                write(c + LOOKAHEAD - NBUF, g_slot).wait()
            gather(c + LOOKAHEAD, g_slot).start()
        gather(c, b).wait()
        write(c, b).start()

    for b in range(NBUF):
        step(b, b, first=(b == 0))

    def group(p, _):
        for b in range(NBUF):
            step(p * NBUF + b, b)
        return _

    lax.fori_loop(1, NCHUNK // NBUF - 1, group, None, unroll=False)

    last = NCHUNK - NBUF
    for b in range(NBUF):
        step(last + b, b, issue=(last + b + LOOKAHEAD < NCHUNK))

    for c in range(NCHUNK - NBUF, NCHUNK):
        write(c, c % NBUF).wait()


def kernel(inputs, embedding_table):
    flat_idx = inputs.reshape(BATCH * SEQ)
    idx4 = (flat_idx[:, None] * SPLIT + jnp.arange(SPLIT, dtype=jnp.int32)).reshape(B)
    table4 = embedding_table.reshape(VOCAB * SPLIT, DS)
    out = _gather_kernel(idx4, table4)
    return out.reshape(BATCH, SEQ, D)


# speedup vs baseline: 1.0652x; 1.0652x over previous
import functools

import jax
import jax.numpy as jnp
from jax import lax
from jax.experimental import pallas as pl
from jax.experimental.pallas import tpu as pltpu
from jax.experimental.pallas import tpu_sc as plsc

VOCAB = 1_000_000
D = 64
BATCH = 4096
SEQ = 50
B = BATCH * SEQ

NC = 2
NS = 16
NW = NC * NS
B_PER_W = B // NW
CHUNK = 64
NCHUNK = B_PER_W // CHUNK
NBUF = 10
LOOKAHEAD = NBUF - 1

_mesh = plsc.VectorSubcoreMesh(
    core_axis_name="c", subcore_axis_name="s", num_cores=NC, num_subcores=NS
)


@functools.partial(
    pl.kernel,
    out_type=jax.ShapeDtypeStruct((B, D), jnp.float32),
    mesh=_mesh,
    scratch_types=[
        pltpu.VMEM((B_PER_W,), jnp.int32),
        pltpu.VMEM((NBUF, CHUNK, D), jnp.float32),
        pltpu.SemaphoreType.DMA((NBUF,)),
        pltpu.SemaphoreType.DMA((NBUF,)),
    ],
    compiler_params=pltpu.CompilerParams(use_tc_tiling_on_sc=False),
)
def _gather_kernel(idx_hbm, table_hbm, out_hbm, idx_v, rows_v, gsem, osem):
    wid = lax.axis_index("s") * NC + lax.axis_index("c")
    base = wid * B_PER_W

    pltpu.sync_copy(idx_hbm.at[pl.ds(base, B_PER_W)], idx_v)

    def gather(c, slot):
        return pltpu.make_async_copy(
            table_hbm.at[idx_v.at[pl.ds(c * CHUNK, CHUNK)]],
            rows_v.at[slot],
            gsem.at[slot],
        )

    def write(c, slot):
        return pltpu.make_async_copy(
            rows_v.at[slot],
            out_hbm.at[pl.ds(base + c * CHUNK, CHUNK)],
            osem.at[slot],
        )

    for c in range(LOOKAHEAD):
        gather(c, c).start()

    def step(c, b, first=False, issue=True):
        if issue:
            g_slot = (b + LOOKAHEAD) % NBUF
            if not first:
                write(c + LOOKAHEAD - NBUF, g_slot).wait()
            gather(c + LOOKAHEAD, g_slot).start()
        gather(c, b).wait()
        write(c, b).start()

    for b in range(NBUF):
        step(b, b, first=(b == 0))

    def group(p, _):
        for b in range(NBUF):
            step(p * NBUF + b, b)
        return _

    lax.fori_loop(1, NCHUNK // NBUF - 1, group, None, unroll=False)

    last = NCHUNK - NBUF
    for b in range(NBUF):
        step(last + b, b, issue=(last + b + LOOKAHEAD < NCHUNK))

    for c in range(NCHUNK - NBUF, NCHUNK):
        write(c, c % NBUF).wait()


def kernel(inputs, embedding_table):
    flat_idx = inputs.reshape(B)
    out = _gather_kernel(flat_idx, embedding_table)
    return out.reshape(BATCH, SEQ, D)
